# Initial kernel scaffold; baseline (speedup 1.0000x reference)
import jax
import jax.numpy as jnp
from jax.experimental import pallas as pl


def kernel(h, adj, w, a_src, a_dst, bias):
    raise NotImplementedError("write your pallas kernel here")


# speedup vs baseline: 1.2883x; 1.2883x over previous
import jax
import jax.numpy as jnp
from jax.experimental import pallas as pl
from jax.experimental.pallas import tpu as pltpu

NEG_SLOPE = 0.2
BM = 256


def _gat_body(h_ref, adj_ref, w_ref, a_ref, b_ref, o_ref,
              hp_ref, src_ref, dst_ref):
    i = pl.program_id(1)
    n, f_out = hp_ref.shape
    bm = adj_ref.shape[1]

    @pl.when(i == 0)
    def _():
        hp = jnp.dot(h_ref[0], w_ref[0], preferred_element_type=jnp.float32)
        hp_ref[...] = hp
        t = jnp.tanh(hp)
        a = a_ref[0]
        src_ref[...] = jax.lax.dot_general(
            t, a[0:1, :], (((1,), (1,)), ((), ())),
            preferred_element_type=jnp.float32)
        dst_ref[...] = jax.lax.dot_general(
            a[1:2, :], t, (((1,), (1,)), ((), ())),
            preferred_element_type=jnp.float32)

    row0 = i * bm
    src_blk = src_ref[pl.ds(row0, bm), :]
    s = src_blk + dst_ref[...]
    s = jnp.where(s >= 0, s, NEG_SLOPE * s)
    rows = jax.lax.broadcasted_iota(jnp.int32, (bm, n), 0) + row0
    cols = jax.lax.broadcasted_iota(jnp.int32, (bm, n), 1)
    keep = (adj_ref[0] != 0) | (rows == cols)
    s = jnp.where(keep, s, -1e30)
    m = jnp.max(s, axis=1, keepdims=True)
    e = jnp.exp(s - m)
    l = jnp.sum(e, axis=1, keepdims=True)
    p = e / l
    o_ref[0, 0] = (jnp.dot(p, hp_ref[...], preferred_element_type=jnp.float32)
                   + b_ref[...])


def kernel(h, adj, w, a_src, a_dst, bias):
    b, n, f_in = h.shape
    hh, _, f_out = w.shape
    a_cat = jnp.concatenate(
        [a_src[:, :, 0][:, None, :], a_dst[:, :, 0][:, None, :]], axis=1)
    adj_i8 = adj.astype(jnp.int8)
    bias2 = bias.reshape(1, f_out)

    grid = (b * hh, n // BM)
    out = pl.pallas_call(
        _gat_body,
        out_shape=jax.ShapeDtypeStruct((b, hh, n, f_out), jnp.float32),
        grid=grid,
        in_specs=[
            pl.BlockSpec((1, n, f_in), lambda g, i: (g // hh, 0, 0)),
            pl.BlockSpec((1, BM, n), lambda g, i: (g // hh, i, 0)),
            pl.BlockSpec((1, f_in, f_out), lambda g, i: (g % hh, 0, 0)),
            pl.BlockSpec((1, 2, f_out), lambda g, i: (g % hh, 0, 0)),
            pl.BlockSpec((1, f_out), lambda g, i: (0, 0)),
        ],
        out_specs=pl.BlockSpec(
            (1, 1, BM, f_out), lambda g, i: (g // hh, g % hh, i, 0)),
        scratch_shapes=[
            pltpu.VMEM((n, f_out), jnp.float32),
            pltpu.VMEM((n, 1), jnp.float32),
            pltpu.VMEM((1, n), jnp.float32),
        ],
        compiler_params=pltpu.CompilerParams(
            dimension_semantics=("parallel", "arbitrary"),
            vmem_limit_bytes=50 * 1024 * 1024,
        ),
        name="fused_graph_attention",
    )(h, adj_i8, w, a_cat, bias2)
    return out


# speedup vs baseline: 1.3234x; 1.0272x over previous
import jax
import jax.numpy as jnp
from jax.experimental import pallas as pl
from jax.experimental.pallas import tpu as pltpu

NEG_SLOPE = 0.2
BM = 256


def _gat_body(h_ref, keep_ref, w_ref, a_ref, b_ref, o_ref,
              hp_ref, src_ref, dst_ref):
    i = pl.program_id(1)
    n, f_out = hp_ref.shape
    bm = keep_ref.shape[1]

    @pl.when(i == 0)
    def _():
        hp = jnp.dot(h_ref[0], w_ref[0], preferred_element_type=jnp.float32)
        hp_ref[...] = hp.astype(jnp.bfloat16)
        t = jnp.tanh(hp)
        a = a_ref[0]
        src_ref[...] = jax.lax.dot_general(
            t, a[0:1, :], (((1,), (1,)), ((), ())),
            preferred_element_type=jnp.float32)
        dst_ref[...] = jax.lax.dot_general(
            a[1:2, :], t, (((1,), (1,)), ((), ())),
            preferred_element_type=jnp.float32)

    row0 = i * bm
    src_blk = src_ref[pl.ds(row0, bm), :]
    s = src_blk + dst_ref[...]
    s = jnp.maximum(s, NEG_SLOPE * s)
    s = jnp.where(keep_ref[0] != 0, s, -1e30)
    m = jnp.max(s, axis=1, keepdims=True)
    e = jnp.exp(s - m)
    l = jnp.sum(e, axis=1, keepdims=True)
    acc = jnp.dot(e.astype(jnp.bfloat16), hp_ref[...],
                  preferred_element_type=jnp.float32)
    o_ref[0, 0] = acc * (1.0 / l) + b_ref[...]


def kernel(h, adj, w, a_src, a_dst, bias):
    b, n, f_in = h.shape
    hh, _, f_out = w.shape
    a_cat = jnp.concatenate(
        [a_src[:, :, 0][:, None, :], a_dst[:, :, 0][:, None, :]], axis=1)
    keep_i8 = (adj | jnp.eye(n, dtype=bool)[None]).astype(jnp.int8)
    h16 = h.astype(jnp.bfloat16)
    w16 = w.astype(jnp.bfloat16)
    bias2 = bias.reshape(1, f_out)

    grid = (b * hh, n // BM)
    out = pl.pallas_call(
        _gat_body,
        out_shape=jax.ShapeDtypeStruct((b, hh, n, f_out), jnp.float32),
        grid=grid,
        in_specs=[
            pl.BlockSpec((1, n, f_in), lambda g, i: (g // hh, 0, 0)),
            pl.BlockSpec((1, BM, n), lambda g, i: (g // hh, i, 0)),
            pl.BlockSpec((1, f_in, f_out), lambda g, i: (g % hh, 0, 0)),
            pl.BlockSpec((1, 2, f_out), lambda g, i: (g % hh, 0, 0)),
            pl.BlockSpec((1, f_out), lambda g, i: (0, 0)),
        ],
        out_specs=pl.BlockSpec(
            (1, 1, BM, f_out), lambda g, i: (g // hh, g % hh, i, 0)),
        scratch_shapes=[
            pltpu.VMEM((n, f_out), jnp.bfloat16),
            pltpu.VMEM((n, 1), jnp.float32),
            pltpu.VMEM((1, n), jnp.float32),
        ],
        compiler_params=pltpu.CompilerParams(
            dimension_semantics=("parallel", "arbitrary"),
            vmem_limit_bytes=50 * 1024 * 1024,
        ),
        name="fused_graph_attention",
    )(h16, keep_i8, w16, a_cat, bias2)
    return out


# speedup vs baseline: 1.4167x; 1.0705x over previous
import jax
import jax.numpy as jnp
from jax.experimental import pallas as pl
from jax.experimental.pallas import tpu as pltpu

NEG_SLOPE = 0.2
BM = 1024
BC = 256


def _gat_body(h_ref, keep_ref, w_ref, a_ref, b_ref, o_ref,
              hp_ref, src_ref, dst_ref):
    i = pl.program_id(1)
    n, f_out = hp_ref.shape
    bm = keep_ref.shape[1]

    @pl.when(i == 0)
    def _():
        hp = jnp.dot(h_ref[0], w_ref[0], preferred_element_type=jnp.float32)
        hp_ref[...] = hp.astype(jnp.bfloat16)
        t = jnp.tanh(hp)
        a = a_ref[0]
        src_ref[...] = jax.lax.dot_general(
            t, a[0:1, :], (((1,), (1,)), ((), ())),
            preferred_element_type=jnp.float32)
        dst_ref[...] = jax.lax.dot_general(
            a[1:2, :], t, (((1,), (1,)), ((), ())),
            preferred_element_type=jnp.float32)

    row0 = i * bm
    dst_row = dst_ref[...]
    for r in range(bm // BC):
        c0 = r * BC
        src_blk = src_ref[pl.ds(row0 + c0, BC), :]
        s = src_blk + dst_row
        s = jnp.maximum(s, NEG_SLOPE * s)
        s = jnp.where(keep_ref[0, c0:c0 + BC, :] != 0, s, -1e30)
        m = jnp.max(s, axis=1, keepdims=True)
        e = jnp.exp(s - m)
        l = jnp.sum(e, axis=1, keepdims=True)
        acc = jnp.dot(e.astype(jnp.bfloat16), hp_ref[...],
                      preferred_element_type=jnp.float32)
        o_ref[0, 0, c0:c0 + BC, :] = acc * (1.0 / l) + b_ref[...]


def kernel(h, adj, w, a_src, a_dst, bias):
    b, n, f_in = h.shape
    hh, _, f_out = w.shape
    a_cat = jnp.concatenate(
        [a_src[:, :, 0][:, None, :], a_dst[:, :, 0][:, None, :]], axis=1)
    keep_i8 = (adj | jnp.eye(n, dtype=bool)[None]).astype(jnp.int8)
    h16 = h.astype(jnp.bfloat16)
    w16 = w.astype(jnp.bfloat16)
    bias2 = bias.reshape(1, f_out)

    grid = (b * hh, n // BM)
    out = pl.pallas_call(
        _gat_body,
        out_shape=jax.ShapeDtypeStruct((b, hh, n, f_out), jnp.float32),
        grid=grid,
        in_specs=[
            pl.BlockSpec((1, n, f_in), lambda g, i: (g // hh, 0, 0)),
            pl.BlockSpec((1, BM, n), lambda g, i: (g // hh, i, 0)),
            pl.BlockSpec((1, f_in, f_out), lambda g, i: (g % hh, 0, 0)),
            pl.BlockSpec((1, 2, f_out), lambda g, i: (g % hh, 0, 0)),
            pl.BlockSpec((1, f_out), lambda g, i: (0, 0)),
        ],
        out_specs=pl.BlockSpec(
            (1, 1, BM, f_out), lambda g, i: (g // hh, g % hh, i, 0)),
        scratch_shapes=[
            pltpu.VMEM((n, f_out), jnp.bfloat16),
            pltpu.VMEM((n, 1), jnp.float32),
            pltpu.VMEM((1, n), jnp.float32),
        ],
        compiler_params=pltpu.CompilerParams(
            dimension_semantics=("parallel", "arbitrary"),
            vmem_limit_bytes=50 * 1024 * 1024,
        ),
        name="fused_graph_attention",
    )(h16, keep_i8, w16, a_cat, bias2)
    return out


# speedup vs baseline: 1.4657x; 1.0346x over previous
import jax
import jax.numpy as jnp
from jax.experimental import pallas as pl
from jax.experimental.pallas import tpu as pltpu

NEG_SLOPE = 0.2
BM = 1024
BC = 256


def _gat_body(h_ref, keep_ref, w_ref, a_ref, b_ref, o_ref,
              hp_ref, src_ref, dst_ref):
    i = pl.program_id(1)
    n, f_out = hp_ref.shape
    bm = keep_ref.shape[1]

    @pl.when(i == 0)
    def _():
        hp = jnp.dot(h_ref[0], w_ref[0], preferred_element_type=jnp.float32)
        hp_ref[...] = hp.astype(jnp.bfloat16)
        t = jnp.tanh(hp)
        a = a_ref[0]
        src_ref[...] = jax.lax.dot_general(
            t, a[0:1, :], (((1,), (1,)), ((), ())),
            preferred_element_type=jnp.float32)
        dst_ref[...] = jax.lax.dot_general(
            a[1:2, :], t, (((1,), (1,)), ((), ())),
            preferred_element_type=jnp.float32)

    row0 = i * bm
    dst_row = dst_ref[...]
    for r in range(bm // BC):
        c0 = r * BC
        src_blk = src_ref[pl.ds(row0 + c0, BC), :]
        s = src_blk + dst_row
        s = jnp.maximum(s, NEG_SLOPE * s)
        rows = jax.lax.broadcasted_iota(jnp.int32, (BC, n), 0) + (row0 + c0)
        cols = jax.lax.broadcasted_iota(jnp.int32, (BC, n), 1)
        keep = keep_ref[0, c0:c0 + BC, :] | (rows == cols)
        s = jnp.where(keep, s, -1e30)
        m = jnp.max(s, axis=1, keepdims=True)
        e = jnp.exp(s - m)
        l = jnp.sum(e, axis=1, keepdims=True)
        acc = jnp.dot(e.astype(jnp.bfloat16), hp_ref[...],
                      preferred_element_type=jnp.float32)
        o_ref[0, 0, c0:c0 + BC, :] = acc * (1.0 / l) + b_ref[...]


def kernel(h, adj, w, a_src, a_dst, bias):
    b, n, f_in = h.shape
    hh, _, f_out = w.shape
    a_cat = jnp.concatenate(
        [a_src[:, :, 0][:, None, :], a_dst[:, :, 0][:, None, :]], axis=1)
    bias2 = bias.reshape(1, f_out)

    grid = (b * hh, n // BM)
    out = pl.pallas_call(
        _gat_body,
        out_shape=jax.ShapeDtypeStruct((b, hh, n, f_out), jnp.float32),
        grid=grid,
        in_specs=[
            pl.BlockSpec((1, n, f_in), lambda g, i: (g // hh, 0, 0)),
            pl.BlockSpec((1, BM, n), lambda g, i: (g // hh, i, 0)),
            pl.BlockSpec((1, f_in, f_out), lambda g, i: (g % hh, 0, 0)),
            pl.BlockSpec((1, 2, f_out), lambda g, i: (g % hh, 0, 0)),
            pl.BlockSpec((1, f_out), lambda g, i: (0, 0)),
        ],
        out_specs=pl.BlockSpec(
            (1, 1, BM, f_out), lambda g, i: (g // hh, g % hh, i, 0)),
        scratch_shapes=[
            pltpu.VMEM((n, f_out), jnp.bfloat16),
            pltpu.VMEM((n, 1), jnp.float32),
            pltpu.VMEM((1, n), jnp.float32),
        ],
        compiler_params=pltpu.CompilerParams(
            dimension_semantics=("parallel", "arbitrary"),
            vmem_limit_bytes=58 * 1024 * 1024,
        ),
        name="fused_graph_attention",
    )(h, adj, w, a_cat, bias2)
    return out
